# baseline (speedup 1.0000x reference)
import functools

import jax
import jax.numpy as jnp
from jax import lax
from jax.experimental import pallas as pl
from jax.experimental.pallas import tpu as pltpu
from jax.experimental.pallas import tpu_sc as plsc

_NUM_ELEMENTS = 119
_FEAT_DIM = 92
_D_MODEL = 256
_N_ATOMS = 100000

_VPAD = 128
_FPAD = 128

_NC = 2
_NS = 16
_NW = _NC * _NS
_L = 16

_CHUNK = 80
_NUM_CHUNKS = _N_ATOMS // _CHUNK
_NBUF = 2
_NI = -(-_NUM_CHUNKS // _NW)


def _fuse_body(t_ref, w_ref, b_ref, o_ref):
    o_ref[...] = (
        jnp.dot(t_ref[...], w_ref[...], preferred_element_type=jnp.float32)
        + b_ref[...]
    )


def _fused_table(table, W, b):
    tp = jnp.zeros((_VPAD, _FPAD), jnp.float32).at[:_NUM_ELEMENTS, :_FEAT_DIM].set(table)
    wp = jnp.zeros((_FPAD, _D_MODEL), jnp.float32).at[:_FEAT_DIM].set(W)
    return pl.pallas_call(
        _fuse_body,
        out_shape=jax.ShapeDtypeStruct((_VPAD, _D_MODEL), jnp.float32),
    )(tp, wp, b.reshape(1, _D_MODEL))


_mesh = plsc.VectorSubcoreMesh(
    core_axis_name="c", subcore_axis_name="s", num_cores=_NC, num_subcores=_NS
)


@functools.partial(
    pl.kernel,
    out_type=jax.ShapeDtypeStruct((_N_ATOMS * _D_MODEL,), jnp.float32),
    mesh=_mesh,
    compiler_params=pltpu.CompilerParams(needs_layout_passes=False),
    scratch_types=[
        pltpu.SMEM((_NBUF, _CHUNK), jnp.int32),
        pltpu.VMEM((_CHUNK * _D_MODEL,), jnp.float32),
        pltpu.VMEM((_CHUNK * _D_MODEL,), jnp.float32),
        pltpu.VMEM((_VPAD * _D_MODEL,), jnp.float32),
        pltpu.VMEM_SHARED((_N_ATOMS,), jnp.int32),
    ]
    + [pltpu.SemaphoreType.DMA] * (2 * _NBUF),
)
def _gather(x_hbm, fused_hbm, out_hbm, idx_v, rows0_v, rows1_v, fused_v, x_sh, *sems):
    rows_bufs = (rows0_v, rows1_v)
    isems = sems[0:_NBUF]
    wsems = sems[_NBUF : 2 * _NBUF]
    wid = lax.axis_index("s") * _NC + lax.axis_index("c")

    @pl.when(lax.axis_index("s") == 0)
    def _stage_x():
        pltpu.sync_copy(x_hbm, x_sh)

    pltpu.sync_copy(fused_hbm, fused_v)
    plsc.subcore_barrier()

    def cid(i):
        return wid + i * _NW

    def start_idx(i, p):
        pltpu.async_copy(
            x_sh.at[pl.ds(cid(i) * _CHUNK, _CHUNK)], idx_v.at[p], isems[p]
        )

    def compute_chunk(p):
        rows_flat = rows_bufs[p]

        @plsc.parallel_loop(0, _CHUNK, unroll=8)
        def row(r):
            src0 = idx_v[p, r] * _D_MODEL
            dst0 = r * _D_MODEL
            for c in range(_D_MODEL // _L):
                rows_flat[pl.ds(dst0 + c * _L, _L)] = fused_v[
                    pl.ds(src0 + c * _L, _L)
                ]

    for p in range(_NBUF):
        start_idx(p, p)

    def body(k, carry):
        for p in range(_NBUF):
            i = _NBUF * k + p

            @pl.when(cid(i) < _NUM_CHUNKS)
            def _process():
                pltpu.make_async_copy(
                    x_sh.at[pl.ds(0, _CHUNK)], idx_v.at[p], isems[p]
                ).wait()

                @pl.when(k >= 1)
                def _drain_prev():
                    pltpu.make_async_copy(
                        rows_bufs[p], out_hbm.at[pl.ds(0, _CHUNK * _D_MODEL)],
                        wsems[p],
                    ).wait()

                compute_chunk(p)

                pltpu.async_copy(
                    rows_bufs[p],
                    out_hbm.at[pl.ds(cid(i) * (_CHUNK * _D_MODEL), _CHUNK * _D_MODEL)],
                    wsems[p],
                )

                @pl.when(cid(i + _NBUF) < _NUM_CHUNKS)
                def _prefetch():
                    start_idx(i + _NBUF, p)

        return carry

    lax.fori_loop(0, _NI // _NBUF, body, 0)

    for p in range(_NBUF):
        pltpu.make_async_copy(
            rows_bufs[p], out_hbm.at[pl.ds(0, _CHUNK * _D_MODEL)], wsems[p]
        ).wait()


def kernel(x, table, W, b):
    fused = _fused_table(table, W, b)
    out_flat = _gather(x, fused.reshape(_VPAD * _D_MODEL))
    return out_flat.reshape(_N_ATOMS, _D_MODEL)


# speedup vs baseline: 1.0030x; 1.0030x over previous
import functools

import jax
import jax.numpy as jnp
from jax import lax
from jax.experimental import pallas as pl
from jax.experimental.pallas import tpu as pltpu
from jax.experimental.pallas import tpu_sc as plsc

_NUM_ELEMENTS = 119
_FEAT_DIM = 92
_D_MODEL = 256
_N_ATOMS = 100000

_VPAD = 128
_FPAD = 128

_NC = 2
_NS = 16
_NW = _NC * _NS
_L = 16

_CHUNK = 160
_NUM_CHUNKS = _N_ATOMS // _CHUNK
_NBUF = 2
_NI = -(-_NUM_CHUNKS // _NW)


def _fuse_body(t_ref, w_ref, b_ref, o_ref):
    o_ref[...] = (
        jnp.dot(t_ref[...], w_ref[...], preferred_element_type=jnp.float32)
        + b_ref[...]
    )


def _fused_table(table, W, b):
    tp = jnp.zeros((_VPAD, _FPAD), jnp.float32).at[:_NUM_ELEMENTS, :_FEAT_DIM].set(table)
    wp = jnp.zeros((_FPAD, _D_MODEL), jnp.float32).at[:_FEAT_DIM].set(W)
    return pl.pallas_call(
        _fuse_body,
        out_shape=jax.ShapeDtypeStruct((_VPAD, _D_MODEL), jnp.float32),
    )(tp, wp, b.reshape(1, _D_MODEL))


_mesh = plsc.VectorSubcoreMesh(
    core_axis_name="c", subcore_axis_name="s", num_cores=_NC, num_subcores=_NS
)


@functools.partial(
    pl.kernel,
    out_type=jax.ShapeDtypeStruct((_N_ATOMS * _D_MODEL,), jnp.float32),
    mesh=_mesh,
    compiler_params=pltpu.CompilerParams(needs_layout_passes=False),
    scratch_types=[
        pltpu.SMEM((_NBUF, _CHUNK // 2), jnp.int32),
        pltpu.SMEM((_NBUF, _CHUNK // 2), jnp.int32),
        pltpu.VMEM((_CHUNK * _D_MODEL,), jnp.float32),
        pltpu.VMEM((_CHUNK * _D_MODEL,), jnp.float32),
        pltpu.VMEM((_VPAD * _D_MODEL,), jnp.float32),
        pltpu.VMEM_SHARED((_N_ATOMS,), jnp.int32),
    ]
    + [pltpu.SemaphoreType.DMA] * (2 * _NBUF),
)
def _gather(x_hbm, fused_hbm, out_hbm, idx_a, idx_b, rows0_v, rows1_v, fused_v, x_sh, *sems):
    rows_bufs = (rows0_v, rows1_v)
    idx_bufs = (idx_a, idx_b)
    _H = _CHUNK // 2
    isems = sems[0:_NBUF]
    wsems = sems[_NBUF : 2 * _NBUF]
    wid = lax.axis_index("s") * _NC + lax.axis_index("c")

    @pl.when(lax.axis_index("s") == 0)
    def _stage_x():
        pltpu.sync_copy(x_hbm, x_sh)

    pltpu.sync_copy(fused_hbm, fused_v)
    plsc.subcore_barrier()

    def cid(i):
        return wid + i * _NW

    def start_idx(i, p):
        base = cid(i) * _CHUNK
        pltpu.async_copy(x_sh.at[pl.ds(base, _H)], idx_a.at[p], isems[p])
        pltpu.async_copy(x_sh.at[pl.ds(base + _H, _H)], idx_b.at[p], isems[p])

    def compute_chunk(p):
        rows_flat = rows_bufs[p]
        for h in range(2):
            idx_h = idx_bufs[h]

            @plsc.parallel_loop(0, _H, unroll=4)
            def row(r):
                src0 = idx_h[p, r] * _D_MODEL
                dst0 = (h * _H + r) * _D_MODEL
                for c in range(_D_MODEL // _L):
                    rows_flat[pl.ds(dst0 + c * _L, _L)] = fused_v[
                        pl.ds(src0 + c * _L, _L)
                    ]

    for p in range(_NBUF):
        start_idx(p, p)

    def body(k, carry):
        for p in range(_NBUF):
            i = _NBUF * k + p

            @pl.when(cid(i) < _NUM_CHUNKS)
            def _process():
                pltpu.make_async_copy(
                    x_sh.at[pl.ds(0, _H)], idx_a.at[p], isems[p]
                ).wait()
                pltpu.make_async_copy(
                    x_sh.at[pl.ds(0, _H)], idx_b.at[p], isems[p]
                ).wait()

                @pl.when(k >= 1)
                def _drain_prev():
                    pltpu.make_async_copy(
                        rows_bufs[p], out_hbm.at[pl.ds(0, _CHUNK * _D_MODEL)],
                        wsems[p],
                    ).wait()

                compute_chunk(p)

                pltpu.async_copy(
                    rows_bufs[p],
                    out_hbm.at[pl.ds(cid(i) * (_CHUNK * _D_MODEL), _CHUNK * _D_MODEL)],
                    wsems[p],
                )

                @pl.when(cid(i + _NBUF) < _NUM_CHUNKS)
                def _prefetch():
                    start_idx(i + _NBUF, p)

        return carry

    lax.fori_loop(0, _NI // _NBUF, body, 0)

    for p in range(_NBUF):
        pltpu.make_async_copy(
            rows_bufs[p], out_hbm.at[pl.ds(0, _CHUNK * _D_MODEL)], wsems[p]
        ).wait()


def kernel(x, table, W, b):
    fused = _fused_table(table, W, b)
    out_flat = _gather(x, fused.reshape(_VPAD * _D_MODEL))
    return out_flat.reshape(_N_ATOMS, _D_MODEL)


# speedup vs baseline: 1.0240x; 1.0209x over previous
import functools

import jax
import jax.numpy as jnp
from jax import lax
from jax.experimental import pallas as pl
from jax.experimental.pallas import tpu as pltpu
from jax.experimental.pallas import tpu_sc as plsc

_NUM_ELEMENTS = 119
_FEAT_DIM = 92
_D_MODEL = 256
_N_ATOMS = 100000

_VPAD = 128
_FPAD = 128

_NC = 2
_NS = 16
_NW = _NC * _NS
_L = 16

_CHUNK = 80
_NUM_CHUNKS = _N_ATOMS // _CHUNK
_NBUF = 2
_NI = -(-_NUM_CHUNKS // _NW)


def _fuse_body(t_ref, w_ref, b_ref, o_ref):
    o_ref[...] = (
        jnp.dot(t_ref[...], w_ref[...], preferred_element_type=jnp.float32)
        + b_ref[...]
    )


def _fused_table(table, W, b):
    tp = jnp.zeros((_VPAD, _FPAD), jnp.float32).at[:_NUM_ELEMENTS, :_FEAT_DIM].set(table)
    wp = jnp.zeros((_FPAD, _D_MODEL), jnp.float32).at[:_FEAT_DIM].set(W)
    return pl.pallas_call(
        _fuse_body,
        out_shape=jax.ShapeDtypeStruct((_VPAD, _D_MODEL), jnp.float32),
    )(tp, wp, b.reshape(1, _D_MODEL))


_mesh = plsc.VectorSubcoreMesh(
    core_axis_name="c", subcore_axis_name="s", num_cores=_NC, num_subcores=_NS
)


@functools.partial(
    pl.kernel,
    out_type=jax.ShapeDtypeStruct((_N_ATOMS * _D_MODEL,), jnp.float32),
    mesh=_mesh,
    compiler_params=pltpu.CompilerParams(needs_layout_passes=False),
    scratch_types=[
        pltpu.SMEM((_NBUF, _CHUNK), jnp.int32),
        pltpu.VMEM((_VPAD * _D_MODEL,), jnp.float32),
        pltpu.VMEM_SHARED((_N_ATOMS,), jnp.int32),
    ]
    + [pltpu.SemaphoreType.DMA] * (_NBUF + 1),
)
def _gather(x_hbm, fused_hbm, out_hbm, idx_v, fused_v, x_sh, *sems):
    isems = sems[0:_NBUF]
    wsem = sems[_NBUF]
    wid = lax.axis_index("s") * _NC + lax.axis_index("c")

    @pl.when(lax.axis_index("s") == 0)
    def _stage_x():
        pltpu.sync_copy(x_hbm, x_sh)

    pltpu.sync_copy(fused_hbm, fused_v)
    plsc.subcore_barrier()

    def cid(i):
        return wid + i * _NW

    def start_idx(i, p):
        pltpu.async_copy(
            x_sh.at[pl.ds(cid(i) * _CHUNK, _CHUNK)], idx_v.at[p], isems[p]
        )

    def compute_chunk(i, p):
        dst0 = cid(i) * (_CHUNK * _D_MODEL)

        def row(r, carry):
            src = idx_v[p, r] * _D_MODEL
            pltpu.async_copy(
                fused_v.at[pl.ds(src, _D_MODEL)],
                out_hbm.at[pl.ds(dst0 + r * _D_MODEL, _D_MODEL)],
                wsem,
            )
            return carry

        lax.fori_loop(0, _CHUNK, row, 0)

    for p in range(_NBUF):
        start_idx(p, p)

    def body(k, carry):
        for p in range(_NBUF):
            i = _NBUF * k + p

            @pl.when(cid(i) < _NUM_CHUNKS)
            def _process():
                pltpu.make_async_copy(
                    x_sh.at[pl.ds(0, _CHUNK)], idx_v.at[p], isems[p]
                ).wait()

                compute_chunk(i, p)

                @pl.when(cid(i + _NBUF) < _NUM_CHUNKS)
                def _prefetch():
                    start_idx(i + _NBUF, p)

        return carry

    lax.fori_loop(0, _NI // _NBUF, body, 0)

    def drain(k, carry):
        pltpu.make_async_copy(
            fused_v.at[pl.ds(0, _CHUNK * _D_MODEL)],
            out_hbm.at[pl.ds(0, _CHUNK * _D_MODEL)],
            wsem,
        ).wait()
        return carry

    n_mine = (_NUM_CHUNKS - 1 - wid) // _NW + 1
    lax.fori_loop(0, n_mine, drain, 0)


def kernel(x, table, W, b):
    fused = _fused_table(table, W, b)
    out_flat = _gather(x, fused.reshape(_VPAD * _D_MODEL))
    return out_flat.reshape(_N_ATOMS, _D_MODEL)


# speedup vs baseline: 1.0278x; 1.0037x over previous
import functools

import jax
import jax.numpy as jnp
from jax import lax
from jax.experimental import pallas as pl
from jax.experimental.pallas import tpu as pltpu
from jax.experimental.pallas import tpu_sc as plsc

_NUM_ELEMENTS = 119
_FEAT_DIM = 92
_D_MODEL = 256
_N_ATOMS = 100000

_VPAD = 128
_FPAD = 128

_NC = 2
_NS = 16
_NW = _NC * _NS
_L = 16

_CHUNK = 80
_NUM_CHUNKS = _N_ATOMS // _CHUNK
_NBUF = 2
_NI = -(-_NUM_CHUNKS // _NW)


def _fuse_body(t_ref, w_ref, b_ref, o_ref):
    o_ref[...] = (
        jnp.dot(t_ref[...], w_ref[...], preferred_element_type=jnp.float32)
        + b_ref[...]
    )


def _fused_table(table, W, b):
    tp = jnp.zeros((_VPAD, _FPAD), jnp.float32).at[:_NUM_ELEMENTS, :_FEAT_DIM].set(table)
    wp = jnp.zeros((_FPAD, _D_MODEL), jnp.float32).at[:_FEAT_DIM].set(W)
    return pl.pallas_call(
        _fuse_body,
        out_shape=jax.ShapeDtypeStruct((_VPAD, _D_MODEL), jnp.float32),
    )(tp, wp, b.reshape(1, _D_MODEL))


_mesh = plsc.VectorSubcoreMesh(
    core_axis_name="c", subcore_axis_name="s", num_cores=_NC, num_subcores=_NS
)


@functools.partial(
    pl.kernel,
    out_type=jax.ShapeDtypeStruct((_N_ATOMS * _D_MODEL,), jnp.float32),
    mesh=_mesh,
    compiler_params=pltpu.CompilerParams(needs_layout_passes=False),
    scratch_types=[
        pltpu.SMEM((_NBUF, _CHUNK), jnp.int32),
        pltpu.VMEM((_CHUNK * _D_MODEL,), jnp.float32),
        pltpu.VMEM((_CHUNK * _D_MODEL,), jnp.float32),
        pltpu.VMEM((_VPAD * _D_MODEL,), jnp.float32),
        pltpu.VMEM_SHARED((_N_ATOMS,), jnp.int32),
        pltpu.VMEM_SHARED((_VPAD * _D_MODEL,), jnp.float32),
        pltpu.VMEM((6256,), jnp.int32),
    ]
    + [pltpu.SemaphoreType.DMA] * (2 * _NBUF),
)
def _gather(x_hbm, fused_hbm, out_hbm, idx_v, rows0_v, rows1_v, fused_v, x_sh, fused_sh, xtmp_v, *sems):
    rows_bufs = (rows0_v, rows1_v)
    isems = sems[0:_NBUF]
    wsems = sems[_NBUF : 2 * _NBUF]
    wid = lax.axis_index("s") * _NC + lax.axis_index("c")

    sid = lax.axis_index("s")
    _XS = 6256
    _XL = _N_ATOMS - 15 * _XS

    @pl.when(sid == 0)
    def _stage_fused_sh():
        pltpu.sync_copy(fused_hbm, fused_sh)

    @pl.when(sid < 15)
    def _stage_x_head():
        base = sid * _XS
        pltpu.sync_copy(x_hbm.at[pl.ds(base, _XS)], xtmp_v)
        pltpu.sync_copy(xtmp_v, x_sh.at[pl.ds(base, _XS)])

    @pl.when(sid == 15)
    def _stage_x_tail():
        pltpu.sync_copy(x_hbm.at[pl.ds(15 * _XS, _XL)], xtmp_v.at[pl.ds(0, _XL)])
        pltpu.sync_copy(xtmp_v.at[pl.ds(0, _XL)], x_sh.at[pl.ds(15 * _XS, _XL)])

    plsc.subcore_barrier()
    pltpu.sync_copy(fused_sh, fused_v)
    plsc.subcore_barrier()

    def cid(i):
        return wid + i * _NW

    def start_idx(i, p):
        pltpu.async_copy(
            x_sh.at[pl.ds(cid(i) * _CHUNK, _CHUNK)], idx_v.at[p], isems[p]
        )

    def compute_chunk(p):
        rows_flat = rows_bufs[p]

        @plsc.parallel_loop(0, _CHUNK, unroll=2)
        def row(r):
            src0 = idx_v[p, r] * _D_MODEL
            dst0 = r * _D_MODEL
            vals = [
                fused_v[pl.ds(src0 + c * _L, _L)]
                for c in range(_D_MODEL // _L)
            ]
            for c in range(_D_MODEL // _L):
                rows_flat[pl.ds(dst0 + c * _L, _L)] = vals[c]

    for p in range(_NBUF):
        start_idx(p, p)

    def body(k, carry):
        for p in range(_NBUF):
            i = _NBUF * k + p

            @pl.when(cid(i) < _NUM_CHUNKS)
            def _process():
                pltpu.make_async_copy(
                    x_sh.at[pl.ds(0, _CHUNK)], idx_v.at[p], isems[p]
                ).wait()

                @pl.when(k >= 1)
                def _drain_prev():
                    pltpu.make_async_copy(
                        rows_bufs[p], out_hbm.at[pl.ds(0, _CHUNK * _D_MODEL)],
                        wsems[p],
                    ).wait()

                compute_chunk(p)

                pltpu.async_copy(
                    rows_bufs[p],
                    out_hbm.at[pl.ds(cid(i) * (_CHUNK * _D_MODEL), _CHUNK * _D_MODEL)],
                    wsems[p],
                )

                @pl.when(cid(i + _NBUF) < _NUM_CHUNKS)
                def _prefetch():
                    start_idx(i + _NBUF, p)

        return carry

    lax.fori_loop(0, _NI // _NBUF, body, 0)

    for p in range(_NBUF):
        pltpu.make_async_copy(
            rows_bufs[p], out_hbm.at[pl.ds(0, _CHUNK * _D_MODEL)], wsems[p]
        ).wait()


def kernel(x, table, W, b):
    fused = _fused_table(table, W, b)
    out_flat = _gather(x, fused.reshape(_VPAD * _D_MODEL))
    return out_flat.reshape(_N_ATOMS, _D_MODEL)
